# Initial kernel scaffold; baseline (speedup 1.0000x reference)
import jax
import jax.numpy as jnp
from jax.experimental import pallas as pl


def kernel(x, edge_index, W_l, b_l, W_r, Ws, Wt):
    raise NotImplementedError("write your pallas kernel here")


# speedup vs baseline: 4.4003x; 4.4003x over previous
import functools

import jax
import jax.numpy as jnp
from jax import lax
from jax.experimental import pallas as pl
from jax.experimental.pallas import tpu as pltpu
from jax.experimental.pallas import tpu_sc as plsc

_N = 10000
_E = 320000
_D = 128

_NC = 2
_NS = 16
_EPW = _E // (_NC * _NS)
_CH = 80
_NCHUNK = _EPW // _CH
_RPT = 624
_TAIL = _N - _NS * _RPT

_ROW_BLK = 1000
_GRID = _N // _ROW_BLK


def _pre_body(x_ref, wr_ref, bl_ref, h_ref, hr_ref):
    h = jnp.maximum(x_ref[...], 0.0)
    h_ref[...] = h
    hr_ref[...] = (
        jnp.dot(h, wr_ref[...], preferred_element_type=jnp.float32,
                precision=lax.Precision.HIGHEST)
        + bl_ref[...]
    )


def _pre(x, W_r, b_l2):
    return pl.pallas_call(
        _pre_body,
        grid=(_GRID,),
        in_specs=[
            pl.BlockSpec((_ROW_BLK, _D), lambda i: (i, 0)),
            pl.BlockSpec((_D, _D), lambda i: (0, 0)),
            pl.BlockSpec((1, _D), lambda i: (0, 0)),
        ],
        out_specs=[
            pl.BlockSpec((_ROW_BLK, _D), lambda i: (i, 0)),
            pl.BlockSpec((_ROW_BLK, _D), lambda i: (i, 0)),
        ],
        out_shape=[
            jax.ShapeDtypeStruct((_N, _D), jnp.float32),
            jax.ShapeDtypeStruct((_N, _D), jnp.float32),
        ],
    )(x, W_r, b_l2)


@functools.partial(
    pl.kernel,
    mesh=plsc.VectorSubcoreMesh(core_axis_name="c", subcore_axis_name="s"),
    out_type=[
        jax.ShapeDtypeStruct((_NC, _N, _D), jnp.float32),
        jax.ShapeDtypeStruct((_NC, _N, _D), jnp.float32),
    ],
    scratch_types=[
        pltpu.VMEM((_CH,), jnp.int32),
        pltpu.VMEM((_CH,), jnp.int32),
        pltpu.VMEM((_CH, _D), jnp.float32),
        pltpu.VMEM((_CH, _D), jnp.float32),
        pltpu.VMEM_SHARED((_N, _D), jnp.float32),
        pltpu.SemaphoreType.DMA,
    ],
)
def _sc_agg(src_hbm, dst_hbm, h_hbm, zero_hbm, ones_hbm,
            agg_out, cnt_out, src_v, dst_v, rows_v, ones_v, sh_acc, sem):
    cid = lax.axis_index("c")
    sid = lax.axis_index("s")
    wid = cid * _NS + sid

    pltpu.sync_copy(ones_hbm, ones_v)

    @pl.when(sid == 0)
    def _init_a():
        pltpu.sync_copy(zero_hbm, sh_acc)

    plsc.subcore_barrier()

    def body_a(j, carry):
        base = wid * _EPW + j * _CH
        pltpu.sync_copy(src_hbm.at[pl.ds(base, _CH)], src_v)
        pltpu.sync_copy(dst_hbm.at[pl.ds(base, _CH)], dst_v)
        pltpu.async_copy(h_hbm.at[src_v], rows_v, sem).wait()
        pltpu.sync_copy(rows_v, sh_acc.at[dst_v], add=True)
        return carry

    lax.fori_loop(0, _NCHUNK, body_a, 0)
    plsc.subcore_barrier()

    pltpu.sync_copy(sh_acc.at[pl.ds(sid * _RPT, _RPT)],
                    agg_out.at[cid, pl.ds(sid * _RPT, _RPT)])

    @pl.when(sid == _NS - 1)
    def _drain_a_tail():
        pltpu.sync_copy(sh_acc.at[pl.ds(_NS * _RPT, _TAIL)],
                        agg_out.at[cid, pl.ds(_NS * _RPT, _TAIL)])

    plsc.subcore_barrier()

    @pl.when(sid == 0)
    def _init_b():
        pltpu.sync_copy(zero_hbm, sh_acc)

    plsc.subcore_barrier()

    def body_b(j, carry):
        base = wid * _EPW + j * _CH
        pltpu.sync_copy(dst_hbm.at[pl.ds(base, _CH)], dst_v)
        pltpu.sync_copy(ones_v, sh_acc.at[dst_v], add=True)
        return carry

    lax.fori_loop(0, _NCHUNK, body_b, 0)
    plsc.subcore_barrier()

    pltpu.sync_copy(sh_acc.at[pl.ds(sid * _RPT, _RPT)],
                    cnt_out.at[cid, pl.ds(sid * _RPT, _RPT)])

    @pl.when(sid == _NS - 1)
    def _drain_b_tail():
        pltpu.sync_copy(sh_acc.at[pl.ds(_NS * _RPT, _TAIL)],
                        cnt_out.at[cid, pl.ds(_NS * _RPT, _TAIL)])


def _post_body(aggp_ref, cntp_ref, hr_ref, wl_ref, ws_ref, wt_ref,
               out1_ref, outs_ref, outt_ref):
    agg = aggp_ref[0] + aggp_ref[1]
    cnt = cntp_ref[0, :, 0:1] + cntp_ref[1, :, 0:1]
    mean = agg / jnp.maximum(cnt, 1.0)
    out1 = (
        jnp.dot(mean, wl_ref[...], preferred_element_type=jnp.float32,
                precision=lax.Precision.HIGHEST)
        + hr_ref[...]
    )
    out1_ref[...] = out1
    zn = out1 / jnp.maximum(
        jnp.sqrt(jnp.sum(out1 * out1, axis=1, keepdims=True)), 1e-12)
    ws = ws_ref[...]
    wsn = ws / jnp.maximum(
        jnp.sqrt(jnp.sum(ws * ws, axis=0, keepdims=True)), 1e-12)
    outs_ref[...] = jnp.dot(zn, wsn, preferred_element_type=jnp.float32,
                            precision=lax.Precision.HIGHEST)
    wt = wt_ref[...]
    wtn = wt / jnp.maximum(
        jnp.sqrt(jnp.sum(wt * wt, axis=0, keepdims=True)), 1e-12)
    outt_ref[...] = jnp.dot(zn, wtn, preferred_element_type=jnp.float32,
                            precision=lax.Precision.HIGHEST)


def _post(aggp, cntp, hr, W_l, Ws, Wt):
    return pl.pallas_call(
        _post_body,
        grid=(_GRID,),
        in_specs=[
            pl.BlockSpec((_NC, _ROW_BLK, _D), lambda i: (0, i, 0)),
            pl.BlockSpec((_NC, _ROW_BLK, _D), lambda i: (0, i, 0)),
            pl.BlockSpec((_ROW_BLK, _D), lambda i: (i, 0)),
            pl.BlockSpec((_D, _D), lambda i: (0, 0)),
            pl.BlockSpec((_D, 40), lambda i: (0, 0)),
            pl.BlockSpec((_D, 100), lambda i: (0, 0)),
        ],
        out_specs=[
            pl.BlockSpec((_ROW_BLK, _D), lambda i: (i, 0)),
            pl.BlockSpec((_ROW_BLK, 40), lambda i: (i, 0)),
            pl.BlockSpec((_ROW_BLK, 100), lambda i: (i, 0)),
        ],
        out_shape=[
            jax.ShapeDtypeStruct((_N, _D), jnp.float32),
            jax.ShapeDtypeStruct((_N, 40), jnp.float32),
            jax.ShapeDtypeStruct((_N, 100), jnp.float32),
        ],
    )(aggp, cntp, hr, W_l, Ws, Wt)


def kernel(x, edge_index, W_l, b_l, W_r, Ws, Wt):
    src = edge_index[0]
    dst = edge_index[1]
    h, hr = _pre(x, W_r, b_l.reshape(1, _D))
    zero = jnp.zeros((_N, _D), dtype=jnp.float32)
    ones = jnp.ones((_CH, _D), dtype=jnp.float32)
    aggp, cntp = _sc_agg(src, dst, h, zero, ones)
    out1, out_s, out_t = _post(aggp, cntp, hr, W_l, Ws, Wt)
    return (out1, out_s, out_t)
